# baseline (speedup 1.0000x reference)
import functools

import jax
import jax.numpy as jnp
from jax import lax
from jax.experimental import pallas as pl
from jax.experimental.pallas import tpu as pltpu
from jax.experimental.pallas import tpu_sc as plsc

N = 10000
E = 320000
D = 128
NAG = 1024
EPS = 1e-5

NC = 2
NS = 16
NW = NC * NS
EPT = E // NW
NPAD = 10240
NROW = 10016
CH = 2000
SLOTS_PER_TILE = NAG // NS

_mesh = plsc.VectorSubcoreMesh(core_axis_name="c", subcore_axis_name="s")
_sc_params = pltpu.CompilerParams(needs_layout_passes=False)


@functools.partial(
    pl.kernel,
    out_type=jax.ShapeDtypeStruct((NW, NPAD), jnp.float32),
    mesh=_mesh,
    compiler_params=_sc_params,
    scratch_types=[
        pltpu.VMEM((EPT,), jnp.int32),
        pltpu.VMEM((NPAD,), jnp.float32),
    ],
)
def _sc_degree(dst_hbm, out_hbm, dst_st, deg_v):
    c = lax.axis_index("c")
    s = lax.axis_index("s")
    wid = s * NC + c
    base = wid * EPT
    pltpu.sync_copy(dst_hbm.at[pl.ds(base, EPT)], dst_st)

    zeros = jnp.zeros((16,), jnp.float32)

    def _zero(i, carry):
        deg_v[pl.ds(i * 16, 16)] = zeros
        return carry

    lax.fori_loop(0, NPAD // 16, _zero, 0, unroll=8)

    ones = jnp.ones((16,), jnp.float32)

    def _count(i, carry):
        d = dst_st[pl.ds(i * 16, 16)]
        plsc.addupdate_scatter(deg_v, [d], ones)
        return carry

    lax.fori_loop(0, EPT // 16, _count, 0, unroll=4)
    pltpu.sync_copy(deg_v, out_hbm.at[wid])


def _tc_dinv_body(deg_ref, dinv_ref):
    s = jnp.sum(deg_ref[...], axis=0)
    dinv_ref[...] = lax.rsqrt(s + 1.0)


def _tc_dinv(deg_parts):
    return pl.pallas_call(
        _tc_dinv_body,
        out_shape=jax.ShapeDtypeStruct((NPAD // 128, 128), jnp.float32),
    )(deg_parts.reshape(NW, NPAD // 128, 128))


def _tc_scale_body(x_ref, dcol_ref, y_ref):
    y_ref[...] = x_ref[...] * dcol_ref[...]


def _tc_scale(x, dinv_col):
    return pl.pallas_call(
        _tc_scale_body,
        out_shape=jax.ShapeDtypeStruct((N, D), jnp.float32),
    )(x, dinv_col)


NROW2 = 2048
DUMMY = NAG
GRP = 128


@functools.partial(
    pl.kernel,
    out_type=(
        jax.ShapeDtypeStruct((NC, NAG, D), jnp.float32),
        jax.ShapeDtypeStruct((NAG, D), jnp.float32),
        jax.ShapeDtypeStruct((NAG,), jnp.float32),
    ),
    mesh=_mesh,
    compiler_params=_sc_params,
    scratch_types=[
        pltpu.VMEM_SHARED((NROW2, D), jnp.float32),
        pltpu.VMEM((EPT + 544,), jnp.int32),
        pltpu.VMEM((EPT + 544,), jnp.int32),
        pltpu.VMEM((88, GRP), jnp.int32),
        pltpu.VMEM((NPAD,), jnp.int32),
        pltpu.VMEM((GRP, D), jnp.float32),
        pltpu.VMEM((GRP, D), jnp.float32),
        pltpu.VMEM((64, D), jnp.float32),
        pltpu.VMEM((NAG,), jnp.int32),
        pltpu.VMEM((NPAD,), jnp.float32),
        pltpu.VMEM((SLOTS_PER_TILE,), jnp.float32),
        pltpu.SemaphoreType.DMA,
    ],
)
def _sc_aggregate(src_hbm, dst_hbm, y_hbm, am_hbm, dinv_hbm,
                  acc_out, yrow_out, dinv_out,
                  acc_sh, src_st, dst_st, rnk2d, rank_st,
                  buf_a, buf_b, zero_v, am_st, dinv_st, dv_st, sem):
    c = lax.axis_index("c")
    s = lax.axis_index("s")
    wid = s * NC + c
    base = wid * EPT

    zeros = jnp.zeros((16,), jnp.float32)

    def _zero2(i, carry):
        r = i // 8
        k = i % 8
        zero_v[r, pl.ds(k * 16, 16)] = zeros
        return carry

    lax.fori_loop(0, 64 * D // 16, _zero2, 0, unroll=8)

    for j in range(2):
        pltpu.sync_copy(zero_v, acc_sh.at[pl.ds(s * 128 + j * 64, 64)])
    plsc.subcore_barrier()

    with jax.named_scope("ph0_stage"):
        pltpu.sync_copy(dinv_hbm, dinv_st)
        pltpu.sync_copy(am_hbm, am_st)
        pltpu.sync_copy(src_hbm.at[pl.ds(base, EPT)], src_st.at[pl.ds(0, EPT)])
        pltpu.sync_copy(dst_hbm.at[pl.ds(base, EPT)], dst_st.at[pl.ds(0, EPT)])

    dummies = jnp.full((16,), DUMMY, jnp.int32)

    def _zr(i, carry):
        rank_st[pl.ds(i * 16, 16)] = dummies
        return carry

    with jax.named_scope("ph1_rankinit"):
        lax.fori_loop(0, NPAD // 16, _zr, 0, unroll=8)

    lane = lax.iota(jnp.int32, 16)

    def _sr(i, carry):
        ids = am_st[pl.ds(i * 16, 16)]
        plsc.store_scatter(rank_st, [ids], lane + i * 16)
        return carry

    lax.fori_loop(0, NAG // 16, _sr, 0, unroll=4)

    def _cmp(i, fill):
        s16 = src_st[pl.ds(i * 16, 16)]
        d16 = dst_st[pl.ds(i * 16, 16)]
        r16 = plsc.load_gather(rank_st, [d16])
        m = r16 < NAG
        plsc.store_compressed(src_st.at[pl.ds(fill, 16)], s16, mask=m)
        plsc.store_compressed(dst_st.at[pl.ds(fill, 16)], r16, mask=m)
        return fill + plsc.all_reduce_population_count(m)[0]

    with jax.named_scope("ph2_compact"):
        fill = lax.fori_loop(0, EPT // 16, _cmp, jnp.int32(0), unroll=2)

    zeros_i = jnp.zeros((16,), jnp.int32)
    for t in range(32):
        src_st[pl.ds(fill + t * 16, 16)] = zeros_i
        dst_st[pl.ds(fill + t * 16, 16)] = dummies

    ngrp = (fill + GRP - 1) // GRP
    ngrp2 = ((ngrp + 1) // 2) * 2
    nb = ngrp2 // 2

    def _r2d(g, carry):
        for k in range(GRP // 16):
            rnk2d[g, pl.ds(k * 16, 16)] = dst_st[pl.ds(g * GRP + k * 16, 16)]
        return carry

    lax.fori_loop(0, ngrp2, _r2d, 0)

    def _gather(g, buf):
        return pltpu.async_copy(
            y_hbm.at[src_st.at[pl.ds(g * GRP, GRP)]], buf, sem)

    with jax.named_scope("ph3_edges"):
        _gather(0, buf_a)

        def _body(j, carry):
            g = j * 2
            _gather(g + 1, buf_b)
            pltpu.make_async_copy(
                y_hbm.at[src_st.at[pl.ds(g * GRP, GRP)]], buf_a, sem).wait()
            pltpu.sync_copy(buf_a, acc_sh.at[rnk2d.at[g]], add=True)
            _gather(g + 2, buf_a)
            pltpu.make_async_copy(
                y_hbm.at[src_st.at[pl.ds(g * GRP, GRP)]], buf_b, sem).wait()
            pltpu.sync_copy(buf_b, acc_sh.at[rnk2d.at[g + 1]], add=True)
            return carry

        lax.fori_loop(0, nb, _body, 0)
        pltpu.make_async_copy(
            y_hbm.at[src_st.at[pl.ds(0, GRP)]], buf_a, sem).wait()

    plsc.subcore_barrier()

    slot0 = s * SLOTS_PER_TILE
    with jax.named_scope("ph4_slots"):
        for g in range(SLOTS_PER_TILE // 16):
            ids = am_st[pl.ds(slot0 + g * 16, 16)]
            r16 = plsc.load_gather(rank_st, [ids])
            buf = buf_a.at[pl.ds(g * 16, 16)]
            pltpu.sync_copy(acc_sh.at[r16], buf)
            pltpu.sync_copy(buf, acc_out.at[c, pl.ds(slot0 + g * 16, 16)])

    @pl.when(c == 0)
    def _core0_extras():
        for g in range(SLOTS_PER_TILE // 16):
            ids = am_st[pl.ds(slot0 + g * 16, 16)]
            buf = buf_b.at[pl.ds(g * 16, 16)]
            pltpu.async_copy(y_hbm.at[ids], buf, sem).wait()
            pltpu.sync_copy(buf, yrow_out.at[pl.ds(slot0 + g * 16, 16)])
            dv_st[pl.ds(g * 16, 16)] = plsc.load_gather(dinv_st, [ids])
        pltpu.sync_copy(dv_st, dinv_out.at[pl.ds(slot0, SLOTS_PER_TILE)])


def _tc_head_body(acc_ref, yr_ref, dv_ref, wg_ref, bg_ref, w1_ref, b1_ref,
                  g1_ref, e1_ref, w2_ref, b2_ref, g2_ref, e2_ref, wm_ref,
                  bm_ref, out_ref):
    dv = dv_ref[...]
    rows = (acc_ref[0] + acc_ref[1] + yr_ref[...]) * dv
    h = jnp.dot(rows, wg_ref[...], preferred_element_type=jnp.float32)
    h = jnp.maximum(h + bg_ref[...], 0.0)
    z = jnp.dot(h, w1_ref[...], preferred_element_type=jnp.float32)
    z = z + b1_ref[...]
    m = jnp.mean(z, axis=-1, keepdims=True)
    v = jnp.mean((z - m) ** 2, axis=-1, keepdims=True)
    z = (z - m) * lax.rsqrt(v + EPS) * g1_ref[...] + e1_ref[...]
    z = jnp.maximum(z, 0.0)
    z2 = jnp.dot(z, w2_ref[...], preferred_element_type=jnp.float32)
    z2 = z2 + b2_ref[...]
    m = jnp.mean(z2, axis=-1, keepdims=True)
    v = jnp.mean((z2 - m) ** 2, axis=-1, keepdims=True)
    z2 = (z2 - m) * lax.rsqrt(v + EPS) * g2_ref[...] + e2_ref[...]
    z2 = jnp.maximum(z2, 0.0)
    o = jnp.dot(z2, wm_ref[...], preferred_element_type=jnp.float32)
    out_ref[...] = jax.nn.sigmoid(o + bm_ref[...])


def _tc_head(acc, yrows, dinv_col, W_gcn, b_gcn, W1, b1, g1, be1,
             W2, b2, g2, be2, Wmu, bmu):
    return pl.pallas_call(
        _tc_head_body,
        out_shape=jax.ShapeDtypeStruct((NAG, Wmu.shape[1]), jnp.float32),
    )(acc, yrows, dinv_col,
      W_gcn, b_gcn.reshape(1, -1), W1, b1.reshape(1, -1),
      g1.reshape(1, -1), be1.reshape(1, -1), W2, b2.reshape(1, -1),
      g2.reshape(1, -1), be2.reshape(1, -1), Wmu, bmu.reshape(1, -1))


def kernel(x, edge_index, agent_mask, W_gcn, b_gcn, W1, b1, g1, be1,
           W2, b2, g2, be2, Wmu, bmu):
    src = edge_index[0]
    dst = edge_index[1]
    deg_parts = _sc_degree(dst)
    dinv2d = _tc_dinv(deg_parts)
    dinv_flat = dinv2d.reshape(NPAD)
    y = _tc_scale(x, dinv_flat[:N].reshape(N, 1))
    acc, yrows, dinv_ag = _sc_aggregate(src, dst, y, agent_mask, dinv_flat)
    return _tc_head(acc, yrows, dinv_ag.reshape(NAG, 1),
                    W_gcn, b_gcn, W1, b1, g1, be1, W2, b2, g2, be2, Wmu, bmu)


# speedup vs baseline: 3.2851x; 3.2851x over previous
import functools

import jax
import jax.numpy as jnp
from jax import lax
from jax.experimental import pallas as pl
from jax.experimental.pallas import tpu as pltpu
from jax.experimental.pallas import tpu_sc as plsc

N = 10000
E = 320000
D = 128
NAG = 1024
EPS = 1e-5

NC = 2
NS = 16
NW = NC * NS
EPT = E // NW
NPAD = 10240
NROW = 10016
CH = 2000
SLOTS_PER_TILE = NAG // NS

_mesh = plsc.VectorSubcoreMesh(core_axis_name="c", subcore_axis_name="s")
_sc_params = pltpu.CompilerParams(needs_layout_passes=False)


@functools.partial(
    pl.kernel,
    out_type=jax.ShapeDtypeStruct((NW, NPAD), jnp.float32),
    mesh=_mesh,
    compiler_params=_sc_params,
    scratch_types=[
        pltpu.VMEM((EPT,), jnp.int32),
        pltpu.VMEM((NPAD,), jnp.float32),
    ],
)
def _sc_degree(dst_hbm, out_hbm, dst_st, deg_v):
    c = lax.axis_index("c")
    s = lax.axis_index("s")
    wid = s * NC + c
    base = wid * EPT
    pltpu.sync_copy(dst_hbm.at[pl.ds(base, EPT)], dst_st)

    zeros = jnp.zeros((16,), jnp.float32)

    def _zero(i, carry):
        deg_v[pl.ds(i * 16, 16)] = zeros
        return carry

    lax.fori_loop(0, NPAD // 16, _zero, 0, unroll=8)

    ones = jnp.ones((16,), jnp.float32)

    def _count(i, carry):
        d = dst_st[pl.ds(i * 16, 16)]
        plsc.addupdate_scatter(deg_v, [d], ones)
        return carry

    lax.fori_loop(0, EPT // 16, _count, 0, unroll=4)
    pltpu.sync_copy(deg_v, out_hbm.at[wid])


def _tc_dinv_body(deg_ref, dinv_ref):
    s = jnp.sum(deg_ref[...], axis=0)
    dinv_ref[...] = lax.rsqrt(s + 1.0)


def _tc_dinv(deg_parts):
    return pl.pallas_call(
        _tc_dinv_body,
        out_shape=jax.ShapeDtypeStruct((NPAD // 128, 128), jnp.float32),
    )(deg_parts.reshape(NW, NPAD // 128, 128))


def _tc_scale_body(x_ref, dcol_ref, y_ref):
    y_ref[...] = x_ref[...] * dcol_ref[...]


def _tc_scale(x, dinv_col):
    return pl.pallas_call(
        _tc_scale_body,
        out_shape=jax.ShapeDtypeStruct((N, D), jnp.float32),
    )(x, dinv_col)


NROW2 = 2048
DUMMY = NAG
GRP = 128


@functools.partial(
    pl.kernel,
    out_type=(
        jax.ShapeDtypeStruct((NC, NAG, D), jnp.float32),
        jax.ShapeDtypeStruct((NAG, D), jnp.float32),
        jax.ShapeDtypeStruct((NAG,), jnp.float32),
    ),
    mesh=_mesh,
    compiler_params=_sc_params,
    scratch_types=[
        pltpu.VMEM_SHARED((NROW2, D), jnp.float32),
        pltpu.VMEM((EPT + 544,), jnp.int32),
        pltpu.VMEM((EPT + 544,), jnp.int32),
        pltpu.VMEM((88, GRP), jnp.int32),
        pltpu.VMEM((NPAD,), jnp.int32),
        pltpu.VMEM((GRP, D), jnp.float32),
        pltpu.VMEM((GRP, D), jnp.float32),
        pltpu.VMEM((64, D), jnp.float32),
        pltpu.VMEM((NAG,), jnp.int32),
        pltpu.VMEM((NPAD,), jnp.float32),
        pltpu.VMEM((SLOTS_PER_TILE,), jnp.float32),
        pltpu.SemaphoreType.DMA,
    ],
)
def _sc_aggregate(src_hbm, dst_hbm, y_hbm, am_hbm, dinv_hbm,
                  acc_out, yrow_out, dinv_out,
                  acc_sh, src_st, dst_st, rnk2d, rank_st,
                  buf_a, buf_b, zero_v, am_st, dinv_st, dv_st, sem):
    c = lax.axis_index("c")
    s = lax.axis_index("s")
    wid = s * NC + c
    base = wid * EPT

    zeros = jnp.zeros((16,), jnp.float32)

    def _zero2(i, carry):
        r = i // 8
        k = i % 8
        zero_v[r, pl.ds(k * 16, 16)] = zeros
        return carry

    lax.fori_loop(0, 64 * D // 16, _zero2, 0, unroll=8)

    for j in range(2):
        pltpu.sync_copy(zero_v, acc_sh.at[pl.ds(s * 128 + j * 64, 64)])
    plsc.subcore_barrier()

    with jax.named_scope("ph0_stage"):
        pltpu.sync_copy(dinv_hbm, dinv_st)
        pltpu.sync_copy(am_hbm, am_st)
        pltpu.sync_copy(src_hbm.at[pl.ds(base, EPT)], src_st.at[pl.ds(0, EPT)])
        pltpu.sync_copy(dst_hbm.at[pl.ds(base, EPT)], dst_st.at[pl.ds(0, EPT)])

    dummies = jnp.full((16,), DUMMY, jnp.int32)

    def _zr(i, carry):
        rank_st[pl.ds(i * 16, 16)] = dummies
        return carry

    with jax.named_scope("ph1_rankinit"):
        lax.fori_loop(0, NPAD // 16, _zr, 0, unroll=8)

    lane = lax.iota(jnp.int32, 16)

    def _sr(i, carry):
        ids = am_st[pl.ds(i * 16, 16)]
        plsc.store_scatter(rank_st, [ids], lane + i * 16)
        return carry

    lax.fori_loop(0, NAG // 16, _sr, 0, unroll=4)

    def _cmp(i, fill):
        s16 = src_st[pl.ds(i * 16, 16)]
        d16 = dst_st[pl.ds(i * 16, 16)]
        r16 = plsc.load_gather(rank_st, [d16])
        m = r16 < NAG
        plsc.store_compressed(src_st.at[pl.ds(fill, 16)], s16, mask=m)
        plsc.store_compressed(dst_st.at[pl.ds(fill, 16)], r16, mask=m)
        return fill + plsc.all_reduce_population_count(m)[0]

    with jax.named_scope("ph2_compact"):
        fill = lax.fori_loop(0, EPT // 16, _cmp, jnp.int32(0), unroll=2)

    lane16 = lax.iota(jnp.int32, 16)
    for t in range(32):
        spread = (wid * 173 + t * 16 + lane16) % 4096
        src_st[pl.ds(fill + t * 16, 16)] = spread
        dst_st[pl.ds(fill + t * 16, 16)] = dummies

    ngrp = (fill + GRP - 1) // GRP
    ngrp2 = ((ngrp + 1) // 2) * 2
    nb = ngrp2 // 2

    def _r2d(g, carry):
        for k in range(GRP // 16):
            rnk2d[g, pl.ds(k * 16, 16)] = dst_st[pl.ds(g * GRP + k * 16, 16)]
        return carry

    lax.fori_loop(0, ngrp2, _r2d, 0)

    def _gather(g, buf):
        return pltpu.async_copy(
            y_hbm.at[src_st.at[pl.ds(g * GRP, GRP)]], buf, sem)

    with jax.named_scope("ph3_edges"):
        _gather(0, buf_a)

        def _body(j, carry):
            g = j * 2
            _gather(jnp.minimum(g + 1, ngrp2 - 1), buf_b)
            pltpu.make_async_copy(
                y_hbm.at[src_st.at[pl.ds(g * GRP, GRP)]], buf_a, sem).wait()
            pltpu.sync_copy(buf_a, acc_sh.at[rnk2d.at[g]], add=True)
            _gather(jnp.minimum(g + 2, ngrp2 - 1), buf_a)
            pltpu.make_async_copy(
                y_hbm.at[src_st.at[pl.ds(g * GRP, GRP)]], buf_b, sem).wait()
            pltpu.sync_copy(buf_b, acc_sh.at[rnk2d.at[g + 1]], add=True)
            return carry

        lax.fori_loop(0, nb, _body, 0)
        pltpu.make_async_copy(
            y_hbm.at[src_st.at[pl.ds(0, GRP)]], buf_a, sem).wait()

    plsc.subcore_barrier()

    slot0 = s * SLOTS_PER_TILE
    with jax.named_scope("ph4_slots"):
        for g in range(SLOTS_PER_TILE // 16):
            ids = am_st[pl.ds(slot0 + g * 16, 16)]
            r16 = plsc.load_gather(rank_st, [ids])
            buf = buf_a.at[pl.ds(g * 16, 16)]
            pltpu.sync_copy(acc_sh.at[r16], buf)
            pltpu.sync_copy(buf, acc_out.at[c, pl.ds(slot0 + g * 16, 16)])

    @pl.when(c == 0)
    def _core0_extras():
        for g in range(SLOTS_PER_TILE // 16):
            ids = am_st[pl.ds(slot0 + g * 16, 16)]
            buf = buf_b.at[pl.ds(g * 16, 16)]
            pltpu.async_copy(y_hbm.at[ids], buf, sem).wait()
            pltpu.sync_copy(buf, yrow_out.at[pl.ds(slot0 + g * 16, 16)])
            dv_st[pl.ds(g * 16, 16)] = plsc.load_gather(dinv_st, [ids])
        pltpu.sync_copy(dv_st, dinv_out.at[pl.ds(slot0, SLOTS_PER_TILE)])


def _tc_head_body(acc_ref, yr_ref, dv_ref, wg_ref, bg_ref, w1_ref, b1_ref,
                  g1_ref, e1_ref, w2_ref, b2_ref, g2_ref, e2_ref, wm_ref,
                  bm_ref, out_ref):
    dv = dv_ref[...]
    rows = (acc_ref[0] + acc_ref[1] + yr_ref[...]) * dv
    h = jnp.dot(rows, wg_ref[...], preferred_element_type=jnp.float32)
    h = jnp.maximum(h + bg_ref[...], 0.0)
    z = jnp.dot(h, w1_ref[...], preferred_element_type=jnp.float32)
    z = z + b1_ref[...]
    m = jnp.mean(z, axis=-1, keepdims=True)
    v = jnp.mean((z - m) ** 2, axis=-1, keepdims=True)
    z = (z - m) * lax.rsqrt(v + EPS) * g1_ref[...] + e1_ref[...]
    z = jnp.maximum(z, 0.0)
    z2 = jnp.dot(z, w2_ref[...], preferred_element_type=jnp.float32)
    z2 = z2 + b2_ref[...]
    m = jnp.mean(z2, axis=-1, keepdims=True)
    v = jnp.mean((z2 - m) ** 2, axis=-1, keepdims=True)
    z2 = (z2 - m) * lax.rsqrt(v + EPS) * g2_ref[...] + e2_ref[...]
    z2 = jnp.maximum(z2, 0.0)
    o = jnp.dot(z2, wm_ref[...], preferred_element_type=jnp.float32)
    out_ref[...] = jax.nn.sigmoid(o + bm_ref[...])


def _tc_head(acc, yrows, dinv_col, W_gcn, b_gcn, W1, b1, g1, be1,
             W2, b2, g2, be2, Wmu, bmu):
    return pl.pallas_call(
        _tc_head_body,
        out_shape=jax.ShapeDtypeStruct((NAG, Wmu.shape[1]), jnp.float32),
    )(acc, yrows, dinv_col,
      W_gcn, b_gcn.reshape(1, -1), W1, b1.reshape(1, -1),
      g1.reshape(1, -1), be1.reshape(1, -1), W2, b2.reshape(1, -1),
      g2.reshape(1, -1), be2.reshape(1, -1), Wmu, bmu.reshape(1, -1))


def kernel(x, edge_index, agent_mask, W_gcn, b_gcn, W1, b1, g1, be1,
           W2, b2, g2, be2, Wmu, bmu):
    src = edge_index[0]
    dst = edge_index[1]
    deg_parts = _sc_degree(dst)
    dinv2d = _tc_dinv(deg_parts)
    dinv_flat = dinv2d.reshape(NPAD)
    y = _tc_scale(x, dinv_flat[:N].reshape(N, 1))
    acc, yrows, dinv_ag = _sc_aggregate(src, dst, y, agent_mask, dinv_flat)
    return _tc_head(acc, yrows, dinv_ag.reshape(NAG, 1),
                    W_gcn, b_gcn, W1, b1, g1, be1, W2, b2, g2, be2, Wmu, bmu)


# speedup vs baseline: 3.6073x; 1.0981x over previous
import functools

import jax
import jax.numpy as jnp
from jax import lax
from jax.experimental import pallas as pl
from jax.experimental.pallas import tpu as pltpu
from jax.experimental.pallas import tpu_sc as plsc

N = 10000
E = 320000
D = 128
NAG = 1024
EPS = 1e-5

NC = 2
NS = 16
NW = NC * NS
EPT = E // NW
NPAD = 10240
NROWS = NPAD // 128
SLOTS_PER_TILE = NAG // NS

NROW2 = 1040
DUMMY = NAG
GRP = 128
SELCAP = EPT + 544

_mesh = plsc.VectorSubcoreMesh(core_axis_name="c", subcore_axis_name="s")
_sc_params = pltpu.CompilerParams(needs_layout_passes=False)


@functools.partial(
    pl.kernel,
    out_type=(
        jax.ShapeDtypeStruct((NC, NAG, D), jnp.float32),
        jax.ShapeDtypeStruct((NAG, D), jnp.float32),
        jax.ShapeDtypeStruct((NAG,), jnp.float32),
    ),
    mesh=_mesh,
    compiler_params=_sc_params,
    scratch_types=[
        pltpu.VMEM_SHARED((NROW2, D), jnp.float32),
        pltpu.VMEM_SHARED((NROWS, 128), jnp.float32),
        pltpu.VMEM((SELCAP,), jnp.int32),
        pltpu.VMEM((SELCAP,), jnp.int32),
        pltpu.VMEM((SELCAP,), jnp.float32),
        pltpu.VMEM((88, GRP), jnp.int32),
        pltpu.VMEM((NPAD,), jnp.int32),
        pltpu.VMEM((NROWS, 128), jnp.float32),
        pltpu.VMEM((GRP, D), jnp.float32),
        pltpu.VMEM((GRP, D), jnp.float32),
        pltpu.VMEM((16, D), jnp.float32),
        pltpu.VMEM((NAG,), jnp.int32),
        pltpu.VMEM((SLOTS_PER_TILE,), jnp.float32),
        pltpu.VMEM((NROWS,), jnp.int32),
        pltpu.SemaphoreType.DMA,
    ],
)
def _sc_fused(src_hbm, dst_hbm, x_hbm, am_hbm,
              acc_out, xrow_out, dinv_out,
              acc_sh, deg_sh, dst_st, src_st, w_st, rnk2d, rank_st,
              deg_v, buf_a, buf_b, zero_v, am_st, dv_st, idx_v, sem):
    c = lax.axis_index("c")
    s = lax.axis_index("s")
    wid = s * NC + c

    zeros = jnp.zeros((16,), jnp.float32)
    lane = lax.iota(jnp.int32, 16)

    def _z16(i, carry):
        zero_v[i // 8, pl.ds((i % 8) * 16, 16)] = zeros
        return carry

    lax.fori_loop(0, 16 * D // 16, _z16, 0, unroll=8)

    for t in range(NROWS // 16):
        idx_v[pl.ds(t * 16, 16)] = lane + t * 16

    for j in range(4):
        pltpu.sync_copy(zero_v, acc_sh.at[pl.ds(s * 65 + j * 16, 16)])
    pltpu.sync_copy(zero_v.at[pl.ds(0, 1)], acc_sh.at[pl.ds(s * 65 + 64, 1)])
    pltpu.sync_copy(zero_v.at[pl.ds(0, 5)], deg_sh.at[pl.ds(s * 5, 5)])

    def _zd(i, carry):
        deg_v[i // 8, pl.ds((i % 8) * 16, 16)] = zeros
        return carry

    lax.fori_loop(0, NPAD // 16, _zd, 0, unroll=8)

    plsc.subcore_barrier()

    ones = jnp.ones((16,), jnp.float32)

    def _count(i, carry):
        d16 = dst_st[pl.ds(i * 16, 16)]
        plsc.addupdate_scatter(
            deg_v, [lax.shift_right_logical(d16, 7), d16 & 127], ones)
        return carry

    half0 = s * 2 * EPT + (1 - c) * EPT
    pltpu.sync_copy(dst_hbm.at[pl.ds(half0, EPT)], dst_st.at[pl.ds(0, EPT)])
    lax.fori_loop(0, EPT // 16, _count, 0, unroll=4)
    half1 = s * 2 * EPT + c * EPT
    pltpu.sync_copy(dst_hbm.at[pl.ds(half1, EPT)], dst_st.at[pl.ds(0, EPT)])
    lax.fori_loop(0, EPT // 16, _count, 0, unroll=4)

    pltpu.sync_copy(deg_v, deg_sh.at[idx_v], add=True)
    plsc.subcore_barrier()

    pltpu.sync_copy(deg_sh.at[pl.ds(s * 5, 5)], deg_v.at[pl.ds(s * 5, 5)])
    magic = jnp.full((16,), 0x5F3759DF, jnp.int32)

    def _newton(b, carry):
        r = s * 5 + b // 8
        col = (b % 8) * 16
        d = deg_v[r, pl.ds(col, 16)] + 1.0
        u = magic - lax.shift_right_logical(plsc.bitcast(d, jnp.int32), 1)
        yv = plsc.bitcast(u, jnp.float32)
        for _ in range(3):
            yv = yv * (1.5 - 0.5 * d * yv * yv)
        deg_v[r, pl.ds(col, 16)] = yv
        return carry

    lax.fori_loop(0, 40, _newton, 0, unroll=4)
    pltpu.sync_copy(deg_v.at[pl.ds(s * 5, 5)], deg_sh.at[pl.ds(s * 5, 5)])
    plsc.subcore_barrier()
    pltpu.sync_copy(deg_sh, deg_v)

    pltpu.sync_copy(am_hbm, am_st)
    dummies = jnp.full((16,), DUMMY, jnp.int32)

    def _zr(i, carry):
        rank_st[pl.ds(i * 16, 16)] = dummies
        return carry

    lax.fori_loop(0, NPAD // 16, _zr, 0, unroll=8)

    def _sr(i, carry):
        ids = am_st[pl.ds(i * 16, 16)]
        plsc.store_scatter(rank_st, [ids], lane + i * 16)
        return carry

    lax.fori_loop(0, NAG // 16, _sr, 0, unroll=4)

    pltpu.sync_copy(src_hbm.at[pl.ds(half1, EPT)], src_st.at[pl.ds(0, EPT)])

    def _cmp(i, fill):
        s16 = src_st[pl.ds(i * 16, 16)]
        d16 = dst_st[pl.ds(i * 16, 16)]
        r16 = plsc.load_gather(rank_st, [d16])
        m = r16 < NAG
        w16 = plsc.load_gather(
            deg_v, [lax.shift_right_logical(s16, 7), s16 & 127])
        plsc.store_compressed(src_st.at[pl.ds(fill, 16)], s16, mask=m)
        plsc.store_compressed(dst_st.at[pl.ds(fill, 16)], r16, mask=m)
        plsc.store_compressed(w_st.at[pl.ds(fill, 16)], w16, mask=m)
        return fill + plsc.all_reduce_population_count(m)[0]

    fill = lax.fori_loop(0, EPT // 16, _cmp, jnp.int32(0), unroll=2)

    for t in range(32):
        src_st[pl.ds(fill + t * 16, 16)] = (wid * 173 + t * 16 + lane) % 4096
        dst_st[pl.ds(fill + t * 16, 16)] = dummies
        w_st[pl.ds(fill + t * 16, 16)] = zeros

    ngrp = (fill + GRP - 1) // GRP
    ngrp2 = ((ngrp + 1) // 2) * 2
    nb = ngrp2 // 2

    def _r2d(g, carry):
        for k in range(GRP // 16):
            rnk2d[g, pl.ds(k * 16, 16)] = dst_st[pl.ds(g * GRP + k * 16, 16)]
        return carry

    lax.fori_loop(0, ngrp2, _r2d, 0)

    def _gather(g, buf):
        return pltpu.async_copy(
            x_hbm.at[src_st.at[pl.ds(g * GRP, GRP)]], buf, sem)

    def _scale(g, buf):
        def _rb(rb, carry):
            for r in range(16):
                off = g * GRP + rb * 16 + r
                wb = plsc.load_gather(w_st, [jnp.full((16,), 0, jnp.int32)
                                             + off])
                row = rb * 16 + r
                for k in range(D // 16):
                    buf[row, pl.ds(k * 16, 16)] = (
                        buf[row, pl.ds(k * 16, 16)] * wb)
            return carry

        lax.fori_loop(0, GRP // 16, _rb, 0)

    _gather(0, buf_a)

    def _body(j, carry):
        g = j * 2
        _gather(jnp.minimum(g + 1, ngrp2 - 1), buf_b)
        pltpu.make_async_copy(
            x_hbm.at[src_st.at[pl.ds(g * GRP, GRP)]], buf_a, sem).wait()
        _scale(g, buf_a)
        pltpu.sync_copy(buf_a, acc_sh.at[rnk2d.at[g]], add=True)
        _gather(jnp.minimum(g + 2, ngrp2 - 1), buf_a)
        pltpu.make_async_copy(
            x_hbm.at[src_st.at[pl.ds(g * GRP, GRP)]], buf_b, sem).wait()
        _scale(g + 1, buf_b)
        pltpu.sync_copy(buf_b, acc_sh.at[rnk2d.at[g + 1]], add=True)
        return carry

    lax.fori_loop(0, nb, _body, 0)
    pltpu.make_async_copy(
        x_hbm.at[src_st.at[pl.ds(0, GRP)]], buf_a, sem).wait()

    plsc.subcore_barrier()

    slot0 = s * SLOTS_PER_TILE
    for g in range(SLOTS_PER_TILE // 16):
        ids = am_st[pl.ds(slot0 + g * 16, 16)]
        r16 = plsc.load_gather(rank_st, [ids])
        buf = buf_a.at[pl.ds(g * 16, 16)]
        pltpu.sync_copy(acc_sh.at[r16], buf)
        pltpu.sync_copy(buf, acc_out.at[c, pl.ds(slot0 + g * 16, 16)])

    @pl.when(c == 0)
    def _core0_extras():
        for g in range(SLOTS_PER_TILE // 16):
            ids = am_st[pl.ds(slot0 + g * 16, 16)]
            buf = buf_b.at[pl.ds(g * 16, 16)]
            pltpu.async_copy(x_hbm.at[ids], buf, sem).wait()
            pltpu.sync_copy(buf, xrow_out.at[pl.ds(slot0 + g * 16, 16)])
            dv_st[pl.ds(g * 16, 16)] = plsc.load_gather(
                deg_v, [lax.shift_right_logical(ids, 7), ids & 127])
        pltpu.sync_copy(dv_st, dinv_out.at[pl.ds(slot0, SLOTS_PER_TILE)])


def _tc_head_body(acc_ref, xr_ref, dv_ref, wg_ref, bg_ref, w1_ref, b1_ref,
                  g1_ref, e1_ref, w2_ref, b2_ref, g2_ref, e2_ref, wm_ref,
                  bm_ref, out_ref):
    dv = dv_ref[...]
    rows = (acc_ref[0] + acc_ref[1]) * dv + xr_ref[...] * dv * dv
    h = jnp.dot(rows, wg_ref[...], preferred_element_type=jnp.float32)
    h = jnp.maximum(h + bg_ref[...], 0.0)
    z = jnp.dot(h, w1_ref[...], preferred_element_type=jnp.float32)
    z = z + b1_ref[...]
    m = jnp.mean(z, axis=-1, keepdims=True)
    v = jnp.mean((z - m) ** 2, axis=-1, keepdims=True)
    z = (z - m) * lax.rsqrt(v + EPS) * g1_ref[...] + e1_ref[...]
    z = jnp.maximum(z, 0.0)
    z2 = jnp.dot(z, w2_ref[...], preferred_element_type=jnp.float32)
    z2 = z2 + b2_ref[...]
    m = jnp.mean(z2, axis=-1, keepdims=True)
    v = jnp.mean((z2 - m) ** 2, axis=-1, keepdims=True)
    z2 = (z2 - m) * lax.rsqrt(v + EPS) * g2_ref[...] + e2_ref[...]
    z2 = jnp.maximum(z2, 0.0)
    o = jnp.dot(z2, wm_ref[...], preferred_element_type=jnp.float32)
    out_ref[...] = jax.nn.sigmoid(o + bm_ref[...])


def _tc_head(acc, xrows, dinv_col, W_gcn, b_gcn, W1, b1, g1, be1,
             W2, b2, g2, be2, Wmu, bmu):
    return pl.pallas_call(
        _tc_head_body,
        out_shape=jax.ShapeDtypeStruct((NAG, Wmu.shape[1]), jnp.float32),
    )(acc, xrows, dinv_col,
      W_gcn, b_gcn.reshape(1, -1), W1, b1.reshape(1, -1),
      g1.reshape(1, -1), be1.reshape(1, -1), W2, b2.reshape(1, -1),
      g2.reshape(1, -1), be2.reshape(1, -1), Wmu, bmu.reshape(1, -1))


def kernel(x, edge_index, agent_mask, W_gcn, b_gcn, W1, b1, g1, be1,
           W2, b2, g2, be2, Wmu, bmu):
    src = edge_index[0]
    dst = edge_index[1]
    acc, xrows, dinv_ag = _sc_fused(src, dst, x, agent_mask)
    return _tc_head(acc, xrows, dinv_ag.reshape(NAG, 1),
                    W_gcn, b_gcn, W1, b1, g1, be1, W2, b2, g2, be2, Wmu, bmu)


# speedup vs baseline: 4.1002x; 1.1366x over previous
import functools

import jax
import jax.numpy as jnp
from jax import lax
from jax.experimental import pallas as pl
from jax.experimental.pallas import tpu as pltpu
from jax.experimental.pallas import tpu_sc as plsc

N = 10000
E = 320000
D = 128
NAG = 1024
EPS = 1e-5

NC = 2
NS = 16
NW = NC * NS
EPT = E // NW
NPAD = 10240
NROWS = NPAD // 128
SLOTS_PER_TILE = NAG // NS

NROW2 = 1040
DUMMY = NAG
GRP = 128
SELCAP = EPT + 544

_mesh = plsc.VectorSubcoreMesh(core_axis_name="c", subcore_axis_name="s")
_sc_params = pltpu.CompilerParams(needs_layout_passes=False)


@functools.partial(
    pl.kernel,
    out_type=(
        jax.ShapeDtypeStruct((NC, NAG, D), jnp.float32),
        jax.ShapeDtypeStruct((NAG, D), jnp.float32),
        jax.ShapeDtypeStruct((NAG,), jnp.float32),
    ),
    mesh=_mesh,
    compiler_params=_sc_params,
    scratch_types=[
        pltpu.VMEM_SHARED((NROW2, D), jnp.float32),
        pltpu.VMEM_SHARED((NROWS, 128), jnp.float32),
        pltpu.VMEM((SELCAP,), jnp.int32),
        pltpu.VMEM((SELCAP,), jnp.int32),
        pltpu.VMEM((SELCAP,), jnp.float32),
        pltpu.VMEM((88, GRP), jnp.int32),
        pltpu.VMEM((NPAD,), jnp.int32),
        pltpu.VMEM((NROWS, 128), jnp.float32),
        pltpu.VMEM((GRP, D), jnp.float32),
        pltpu.VMEM((GRP, D), jnp.float32),
        pltpu.VMEM((16, D), jnp.float32),
        pltpu.VMEM((NAG,), jnp.int32),
        pltpu.VMEM((SLOTS_PER_TILE,), jnp.float32),
        pltpu.VMEM((NROWS,), jnp.int32),
        pltpu.SemaphoreType.DMA,
        pltpu.SemaphoreType.DMA,
    ],
)
def _sc_fused(ef_hbm, x_hbm, am_hbm,
              acc_out, xrow_out, dinv_out,
              acc_sh, deg_sh, dst_st, src_st, w_st, rnk2d, rank_st,
              deg_v, buf_a, buf_b, zero_v, am_st, dv_st, idx_v, sem,
              sem_s):
    c = lax.axis_index("c")
    s = lax.axis_index("s")
    wid = s * NC + c

    zeros = jnp.zeros((16,), jnp.float32)
    lane = lax.iota(jnp.int32, 16)

    def _z16(i, carry):
        zero_v[i // 8, pl.ds((i % 8) * 16, 16)] = zeros
        return carry

    lax.fori_loop(0, 16 * D // 16, _z16, 0, unroll=8)

    for t in range(NROWS // 16):
        idx_v[pl.ds(t * 16, 16)] = lane + t * 16

    for j in range(4):
        pltpu.sync_copy(zero_v, acc_sh.at[pl.ds(s * 65 + j * 16, 16)])
    pltpu.sync_copy(zero_v.at[pl.ds(0, 1)], acc_sh.at[pl.ds(s * 65 + 64, 1)])
    pltpu.sync_copy(zero_v.at[pl.ds(0, 5)], deg_sh.at[pl.ds(s * 5, 5)])

    def _zd(i, carry):
        deg_v[i // 8, pl.ds((i % 8) * 16, 16)] = zeros
        return carry

    lax.fori_loop(0, NPAD // 16, _zd, 0, unroll=8)

    plsc.subcore_barrier()

    ones = jnp.ones((16,), jnp.float32)

    def _count(i, carry):
        d16 = dst_st[pl.ds(i * 16, 16)]
        plsc.addupdate_scatter(
            deg_v, [lax.shift_right_logical(d16, 7), d16 & 127], ones)
        return carry

    half0 = s * 2 * EPT + (1 - c) * EPT
    half1 = s * 2 * EPT + c * EPT
    pltpu.async_copy(ef_hbm.at[pl.ds(half1, EPT)],
                     src_st.at[pl.ds(0, EPT)], sem_s)
    pltpu.sync_copy(ef_hbm.at[pl.ds(E + half0, EPT)],
                    dst_st.at[pl.ds(0, EPT)])
    lax.fori_loop(0, EPT // 16, _count, 0, unroll=4)
    pltpu.sync_copy(ef_hbm.at[pl.ds(E + half1, EPT)],
                    dst_st.at[pl.ds(0, EPT)])
    lax.fori_loop(0, EPT // 16, _count, 0, unroll=4)

    pltpu.sync_copy(deg_v, deg_sh.at[idx_v], add=True)
    plsc.subcore_barrier()

    pltpu.sync_copy(deg_sh.at[pl.ds(s * 5, 5)], deg_v.at[pl.ds(s * 5, 5)])
    magic = jnp.full((16,), 0x5F3759DF, jnp.int32)

    def _newton(b, carry):
        r = s * 5 + b // 8
        col = (b % 8) * 16
        d = deg_v[r, pl.ds(col, 16)] + 1.0
        u = magic - lax.shift_right_logical(plsc.bitcast(d, jnp.int32), 1)
        yv = plsc.bitcast(u, jnp.float32)
        for _ in range(3):
            yv = yv * (1.5 - 0.5 * d * yv * yv)
        deg_v[r, pl.ds(col, 16)] = yv
        return carry

    lax.fori_loop(0, 40, _newton, 0, unroll=4)
    pltpu.sync_copy(deg_v.at[pl.ds(s * 5, 5)], deg_sh.at[pl.ds(s * 5, 5)])
    plsc.subcore_barrier()
    pltpu.sync_copy(deg_sh, deg_v)

    pltpu.sync_copy(am_hbm, am_st)
    dummies = jnp.full((16,), DUMMY, jnp.int32)

    def _zr(i, carry):
        rank_st[pl.ds(i * 16, 16)] = dummies
        return carry

    lax.fori_loop(0, NPAD // 16, _zr, 0, unroll=8)

    def _sr(i, carry):
        ids = am_st[pl.ds(i * 16, 16)]
        plsc.store_scatter(rank_st, [ids], lane + i * 16)
        return carry

    lax.fori_loop(0, NAG // 16, _sr, 0, unroll=4)

    pltpu.make_async_copy(ef_hbm.at[pl.ds(half1, EPT)],
                          src_st.at[pl.ds(0, EPT)], sem_s).wait()

    def _cmp(i, fill):
        s16 = src_st[pl.ds(i * 16, 16)]
        d16 = dst_st[pl.ds(i * 16, 16)]
        r16 = plsc.load_gather(rank_st, [d16])
        m = r16 < NAG
        w16 = plsc.load_gather(
            deg_v, [lax.shift_right_logical(s16, 7), s16 & 127])
        plsc.store_compressed(src_st.at[pl.ds(fill, 16)], s16, mask=m)
        plsc.store_compressed(dst_st.at[pl.ds(fill, 16)], r16, mask=m)
        plsc.store_compressed(w_st.at[pl.ds(fill, 16)], w16, mask=m)
        return fill + plsc.all_reduce_population_count(m)[0]

    fill = lax.fori_loop(0, EPT // 16, _cmp, jnp.int32(0), unroll=2)

    for t in range(32):
        src_st[pl.ds(fill + t * 16, 16)] = (wid * 173 + t * 16 + lane) % 4096
        dst_st[pl.ds(fill + t * 16, 16)] = dummies
        w_st[pl.ds(fill + t * 16, 16)] = zeros

    ngrp = (fill + GRP - 1) // GRP
    ngrp2 = ((ngrp + 1) // 2) * 2
    nb = ngrp2 // 2

    def _r2d(g, carry):
        for k in range(GRP // 16):
            rnk2d[g, pl.ds(k * 16, 16)] = dst_st[pl.ds(g * GRP + k * 16, 16)]
        return carry

    lax.fori_loop(0, ngrp2, _r2d, 0)

    def _gather(g, buf):
        return pltpu.async_copy(
            x_hbm.at[src_st.at[pl.ds(g * GRP, GRP)]], buf, sem)

    def _scale(g, buf):
        def _rb(rb, carry):
            for r in range(16):
                off = g * GRP + rb * 16 + r
                wb = plsc.load_gather(w_st, [jnp.full((16,), 0, jnp.int32)
                                             + off])
                row = rb * 16 + r
                for k in range(D // 16):
                    buf[row, pl.ds(k * 16, 16)] = (
                        buf[row, pl.ds(k * 16, 16)] * wb)
            return carry

        lax.fori_loop(0, GRP // 16, _rb, 0)

    _gather(0, buf_a)

    def _body(j, carry):
        g = j * 2
        _gather(jnp.minimum(g + 1, ngrp2 - 1), buf_b)
        pltpu.make_async_copy(
            x_hbm.at[src_st.at[pl.ds(g * GRP, GRP)]], buf_a, sem).wait()
        _scale(g, buf_a)
        pltpu.sync_copy(buf_a, acc_sh.at[rnk2d.at[g]], add=True)
        _gather(jnp.minimum(g + 2, ngrp2 - 1), buf_a)
        pltpu.make_async_copy(
            x_hbm.at[src_st.at[pl.ds(g * GRP, GRP)]], buf_b, sem).wait()
        _scale(g + 1, buf_b)
        pltpu.sync_copy(buf_b, acc_sh.at[rnk2d.at[g + 1]], add=True)
        return carry

    lax.fori_loop(0, nb, _body, 0)
    pltpu.make_async_copy(
        x_hbm.at[src_st.at[pl.ds(0, GRP)]], buf_a, sem).wait()

    plsc.subcore_barrier()

    slot0 = s * SLOTS_PER_TILE
    for g in range(SLOTS_PER_TILE // 16):
        ids = am_st[pl.ds(slot0 + g * 16, 16)]
        r16 = plsc.load_gather(rank_st, [ids])
        buf = buf_a.at[pl.ds(g * 16, 16)]
        pltpu.sync_copy(acc_sh.at[r16], buf)
        pltpu.sync_copy(buf, acc_out.at[c, pl.ds(slot0 + g * 16, 16)])

    @pl.when(c == 0)
    def _core0_extras():
        for g in range(SLOTS_PER_TILE // 16):
            ids = am_st[pl.ds(slot0 + g * 16, 16)]
            buf = buf_b.at[pl.ds(g * 16, 16)]
            pltpu.async_copy(x_hbm.at[ids], buf, sem).wait()
            pltpu.sync_copy(buf, xrow_out.at[pl.ds(slot0 + g * 16, 16)])
            dv_st[pl.ds(g * 16, 16)] = plsc.load_gather(
                deg_v, [lax.shift_right_logical(ids, 7), ids & 127])
        pltpu.sync_copy(dv_st, dinv_out.at[pl.ds(slot0, SLOTS_PER_TILE)])


def _tc_head_body(acc_ref, xr_ref, dv_ref, wg_ref, bg_ref, w1_ref, b1_ref,
                  g1_ref, e1_ref, w2_ref, b2_ref, g2_ref, e2_ref, wm_ref,
                  bm_ref, out_ref):
    dv = dv_ref[...]
    rows = (acc_ref[0] + acc_ref[1]) * dv + xr_ref[...] * dv * dv
    h = jnp.dot(rows, wg_ref[...], preferred_element_type=jnp.float32)
    h = jnp.maximum(h + bg_ref[...], 0.0)
    z = jnp.dot(h, w1_ref[...], preferred_element_type=jnp.float32)
    z = z + b1_ref[...]
    m = jnp.mean(z, axis=-1, keepdims=True)
    v = jnp.mean((z - m) ** 2, axis=-1, keepdims=True)
    z = (z - m) * lax.rsqrt(v + EPS) * g1_ref[...] + e1_ref[...]
    z = jnp.maximum(z, 0.0)
    z2 = jnp.dot(z, w2_ref[...], preferred_element_type=jnp.float32)
    z2 = z2 + b2_ref[...]
    m = jnp.mean(z2, axis=-1, keepdims=True)
    v = jnp.mean((z2 - m) ** 2, axis=-1, keepdims=True)
    z2 = (z2 - m) * lax.rsqrt(v + EPS) * g2_ref[...] + e2_ref[...]
    z2 = jnp.maximum(z2, 0.0)
    o = jnp.dot(z2, wm_ref[...], preferred_element_type=jnp.float32)
    out_ref[...] = jax.nn.sigmoid(o + bm_ref[...])


def _tc_head(acc, xrows, dinv_col, W_gcn, b_gcn, W1, b1, g1, be1,
             W2, b2, g2, be2, Wmu, bmu):
    return pl.pallas_call(
        _tc_head_body,
        out_shape=jax.ShapeDtypeStruct((NAG, Wmu.shape[1]), jnp.float32),
    )(acc, xrows, dinv_col,
      W_gcn, b_gcn.reshape(1, -1), W1, b1.reshape(1, -1),
      g1.reshape(1, -1), be1.reshape(1, -1), W2, b2.reshape(1, -1),
      g2.reshape(1, -1), be2.reshape(1, -1), Wmu, bmu.reshape(1, -1))


def kernel(x, edge_index, agent_mask, W_gcn, b_gcn, W1, b1, g1, be1,
           W2, b2, g2, be2, Wmu, bmu):
    acc, xrows, dinv_ag = _sc_fused(edge_index.reshape(2 * E), x, agent_mask)
    return _tc_head(acc, xrows, dinv_ag.reshape(NAG, 1),
                    W_gcn, b_gcn, W1, b1, g1, be1, W2, b2, g2, be2, Wmu, bmu)
